# Initial kernel scaffold; baseline (speedup 1.0000x reference)
import jax
import jax.numpy as jnp
from jax.experimental import pallas as pl


def kernel(exposure_params, vignetting_params, color_params, crf_params, rgb, pixel_coords, camera_idcs, frame_idcs):
    raise NotImplementedError("write your pallas kernel here")


# speedup vs baseline: 14.6454x; 14.6454x over previous
import functools

import jax
import jax.numpy as jnp
from jax import lax
from jax.experimental import pallas as pl
from jax.experimental.pallas import tpu as pltpu
from jax.experimental.pallas import tpu_sc as plsc

N = 1048576
NUM_FRAMES = 1000
NUM_CAMERAS = 8
INV_W = 1.0 / 1920.0
INV_H = 1.0 / 1080.0
LN2 = 0.6931471805599453

EXPO_OFF = 0
VIG_OFF = NUM_FRAMES
COL_OFF = VIG_OFF + NUM_CAMERAS * 15
CRF_OFF = COL_OFF + NUM_FRAMES * 8
TAB_LEN = CRF_OFF + NUM_CAMERAS * 12

NC, NS, L = 2, 16, 16
NW = NC * NS
PPW = N // NW
CHUNK = 8192
NCHUNK = PPW // CHUNK
VPC = CHUNK // 16

_LOG2_C = (
    4.12769564e-08, 1.44269483e+00, -7.21361158e-01, 4.80932631e-01,
    -3.59983120e-01, 2.86848207e-01, -2.51114227e-01, 2.36109739e-01,
    -1.41995147e-01,
)


def _log2(x):
    bits = plsc.bitcast(x, jnp.int32)
    e = (bits >> 23) - 127
    m = plsc.bitcast((bits & 0x007FFFFF) | 0x3F800000, jnp.float32)
    big = m > 1.4142135381698608
    m = jnp.where(big, m * 0.5, m)
    e = jnp.where(big, e + 1, e)
    t = m - 1.0
    p = jnp.full((L,), _LOG2_C[8], jnp.float32)
    for c in _LOG2_C[7::-1]:
        p = p * t + c
    return e.astype(jnp.float32) + p


def _ln(x):
    return _log2(x) * LN2


def _kernel_body(tab_hbm, rgb_hbm, xy_hbm, cam_hbm, frm_hbm, out_hbm,
                 tab_v, rgb_v, xy_v, cam_v, frm_v, out_v):
    wid = lax.axis_index("s") * NC + lax.axis_index("c")
    base_px = wid * PPW

    lane = lax.broadcasted_iota(jnp.int32, (L,), 0)

    pltpu.sync_copy(tab_hbm, tab_v)

    k4 = lane & 3
    for i in range(6):
        sl = pl.ds(CRF_OFF + i * L, L)
        t = tab_v[sl]
        sp = jnp.maximum(t, 0.0) + _ln(1.0 + jnp.exp(-jnp.abs(t)))
        val = jnp.where(k4 == 2, (sp + 0.1) * LN2, sp + 0.3)
        tab_v[sl] = jnp.where(k4 == 3, t, val)

    i3 = lane * 3
    i2 = lane * 2

    for ch in range(NCHUNK):
        off = base_px + ch * CHUNK
        pltpu.sync_copy(rgb_hbm.at[pl.ds(off * 3, CHUNK * 3)], rgb_v)
        pltpu.sync_copy(xy_hbm.at[pl.ds(off * 2, CHUNK * 2)], xy_v)
        pltpu.sync_copy(cam_hbm.at[pl.ds(off, CHUNK)], cam_v)
        pltpu.sync_copy(frm_hbm.at[pl.ds(off, CHUNK)], frm_v)

        def vbody(j, carry):
            b16 = j * L
            cam = cam_v[pl.ds(b16, L)]
            frm = frm_v[pl.ds(b16, L)]
            jr = i3 + j * (3 * L)
            jc = i2 + j * (2 * L)

            r = plsc.load_gather(rgb_v, [jr])
            g = plsc.load_gather(rgb_v, [jr + 1])
            b = plsc.load_gather(rgb_v, [jr + 2])
            u = plsc.load_gather(xy_v, [jc]) * INV_W - 0.5
            v = plsc.load_gather(xy_v, [jc + 1]) * INV_H - 0.5

            gain = jnp.exp(plsc.load_gather(tab_v, [frm + EXPO_OFF]))
            r = r * gain
            g = g * gain
            b = b * gain

            vbase = cam * 15 + VIG_OFF
            x3 = []
            for chan, xv in enumerate((r, g, b)):
                o = vbase + chan * 5
                a1 = plsc.load_gather(tab_v, [o])
                a2 = plsc.load_gather(tab_v, [o + 1])
                a3 = plsc.load_gather(tab_v, [o + 2])
                cx = plsc.load_gather(tab_v, [o + 3])
                cy = plsc.load_gather(tab_v, [o + 4])
                du = u - cx
                dv = v - cy
                r2 = du * du + dv * dv
                vig = 1.0 + r2 * (a1 + r2 * (a2 + r2 * a3))
                x3.append(xv * vig)
            xr, xg, xb = x3

            cbase = frm * 8 + COL_OFF
            xr = xr * jnp.exp(plsc.load_gather(tab_v, [cbase]))
            xb = xb * jnp.exp(plsc.load_gather(tab_v, [cbase + 1]))
            m01 = plsc.load_gather(tab_v, [cbase + 2])
            m02 = plsc.load_gather(tab_v, [cbase + 3])
            m10 = plsc.load_gather(tab_v, [cbase + 4])
            m12 = plsc.load_gather(tab_v, [cbase + 5])
            m20 = plsc.load_gather(tab_v, [cbase + 6])
            m21 = plsc.load_gather(tab_v, [cbase + 7])
            ro = (1.0 - m01 - m02) * xr + m01 * xg + m02 * xb
            go = m10 * xr + (1.0 - m10 - m12) * xg + m12 * xb
            bo = m20 * xr + m21 * xg + (1.0 - m20 - m21) * xb

            kbase = cam * 12 + CRF_OFF
            outs = []
            for chan, xv in enumerate((ro, go, bo)):
                o = kbase + chan * 4
                a = plsc.load_gather(tab_v, [o])
                bb = plsc.load_gather(tab_v, [o + 1])
                cl = plsc.load_gather(tab_v, [o + 2])
                d = plsc.load_gather(tab_v, [o + 3])
                xc = jnp.clip(xv, 1e-06, 1.0)
                xc = jnp.exp(cl * _log2(xc))
                y = (bb * a * xc) / ((a - 1.0) * xc + 1.0) + d
                outs.append(y)

            plsc.store_scatter(out_v, [jr], outs[0])
            plsc.store_scatter(out_v, [jr + 1], outs[1])
            plsc.store_scatter(out_v, [jr + 2], outs[2])
            return carry

        lax.fori_loop(0, VPC, vbody, 0)
        pltpu.sync_copy(out_v, out_hbm.at[pl.ds(off * 3, CHUNK * 3)])


@jax.jit
def kernel(exposure_params, vignetting_params, color_params, crf_params,
           rgb, pixel_coords, camera_idcs, frame_idcs):
    tab = jnp.concatenate([
        exposure_params,
        vignetting_params.reshape(-1),
        color_params.reshape(-1),
        crf_params.reshape(-1),
    ])
    mesh = plsc.VectorSubcoreMesh(core_axis_name="c", subcore_axis_name="s")
    run = functools.partial(
        pl.kernel,
        mesh=mesh,
        compiler_params=pltpu.CompilerParams(needs_layout_passes=False),
        out_type=jax.ShapeDtypeStruct((3 * N,), jnp.float32),
        scratch_types=[
            pltpu.VMEM((TAB_LEN,), jnp.float32),
            pltpu.VMEM((CHUNK * 3,), jnp.float32),
            pltpu.VMEM((CHUNK * 2,), jnp.float32),
            pltpu.VMEM((CHUNK,), jnp.int32),
            pltpu.VMEM((CHUNK,), jnp.int32),
            pltpu.VMEM((CHUNK * 3,), jnp.float32),
        ],
    )(_kernel_body)
    y = run(tab, rgb.reshape(-1), pixel_coords.reshape(-1),
            camera_idcs, frame_idcs)
    return y.reshape(N, 3)


# speedup vs baseline: 14.9408x; 1.0202x over previous
import functools

import jax
import jax.numpy as jnp
from jax import lax
from jax.experimental import pallas as pl
from jax.experimental.pallas import tpu as pltpu
from jax.experimental.pallas import tpu_sc as plsc

N = 1048576
NUM_FRAMES = 1000
NUM_CAMERAS = 8
INV_W = 1.0 / 1920.0
INV_H = 1.0 / 1080.0
LN2 = 0.6931471805599453

EXPO_OFF = 0
VIG_OFF = NUM_FRAMES
COL_OFF = VIG_OFF + 15 * NUM_CAMERAS
CRF_OFF = COL_OFF + 8 * NUM_FRAMES
TAB_LEN = CRF_OFF + NUM_CAMERAS * 12
DER_OFF = TAB_LEN
TABV_LEN = DER_OFF + 12 * 16

NC, NS, L = 2, 16, 16
NW = NC * NS
PPW = N // NW
CHUNK = 8192
NCHUNK = PPW // CHUNK
VPC = CHUNK // 16
UNROLL = 4

_LOG2_C = (
    -1.59316620e-06, 1.44271384e+00, -7.21039028e-01, 4.79392296e-01,
    -3.68821683e-01, 3.21648781e-01, -2.00985626e-01,
)
_SQRT2_2_BITS = 0x3F3504F3


def _log2(x):
    bits = plsc.bitcast(x, jnp.int32)
    e = (bits - _SQRT2_2_BITS) >> 23
    m = plsc.bitcast(bits - (e << 23), jnp.float32)
    t = m - 1.0
    p = jnp.full((L,), _LOG2_C[6], jnp.float32)
    for c in _LOG2_C[5::-1]:
        p = p * t + c
    return e.astype(jnp.float32) + p


def _ln(x):
    return _log2(x) * LN2


def _kernel_body(tab_hbm, rgb_hbm, xy_hbm, cam_hbm, frm_hbm, out_hbm,
                 tab_v, rgb_v, xy_v, cam_v, frm_v, out_v):
    wid = lax.axis_index("s") * NC + lax.axis_index("c")
    base_px = wid * PPW

    lane = lax.broadcasted_iota(jnp.int32, (L,), 0)

    pltpu.sync_copy(tab_hbm, tab_v.at[pl.ds(0, TAB_LEN)])

    def tslice(off):
        return tab_v.at[pl.ds(off, TABV_LEN - off)]

    k4 = lane & 3
    for i in range(6):
        sl = pl.ds(CRF_OFF + i * L, L)
        t = tab_v[sl]
        sp = jnp.maximum(t, 0.0) + _ln(1.0 + jnp.exp(-jnp.abs(t)))
        val = jnp.where(k4 == 2, sp + 0.1, sp + 0.3)
        tab_v[sl] = jnp.where(k4 == 3, t, val)

    cam8 = lane & 7
    for a in range(12):
        kind, chan = a // 3, a % 3
        g = (cam8 * 3 + chan) * 4 + CRF_OFF
        if kind == 0:
            val = plsc.load_gather(tab_v, [g]) * plsc.load_gather(tab_v, [g + 1])
        elif kind == 1:
            val = plsc.load_gather(tab_v, [g]) - 1.0
        elif kind == 2:
            val = plsc.load_gather(tab_v, [g + 2]) * LN2
        else:
            val = plsc.load_gather(tab_v, [g + 3])
        tab_v[pl.ds(DER_OFF + a * L, L)] = val

    i3 = lane * 3
    i2 = lane * 2

    for ch in range(NCHUNK):
        off = base_px + ch * CHUNK
        pltpu.sync_copy(rgb_hbm.at[pl.ds(off * 3, CHUNK * 3)], rgb_v)
        pltpu.sync_copy(xy_hbm.at[pl.ds(off * 2, CHUNK * 2)], xy_v)
        pltpu.sync_copy(cam_hbm.at[pl.ds(off, CHUNK)], cam_v)
        pltpu.sync_copy(frm_hbm.at[pl.ds(off, CHUNK)], frm_v)

        @plsc.parallel_loop(0, VPC, step=1, unroll=UNROLL)
        def vbody(j):
            cam = cam_v[pl.ds(j * L, L)]
            frm = frm_v[pl.ds(j * L, L)]
            jr = i3 + j * (3 * L)
            jr1 = jr + 1
            jr2 = jr + 2
            jc = i2 + j * (2 * L)

            r = plsc.load_gather(rgb_v, [jr])
            g = plsc.load_gather(rgb_v, [jr1])
            b = plsc.load_gather(rgb_v, [jr2])
            u = plsc.load_gather(xy_v, [jc]) * INV_W - 0.5
            v = plsc.load_gather(xy_v, [jc + 1]) * INV_H - 0.5

            gain = jnp.exp(plsc.load_gather(tab_v, [frm]))
            r = r * gain
            g = g * gain
            b = b * gain

            x3 = []
            for chan, xv in enumerate((r, g, b)):
                o = VIG_OFF + chan * 5 * 8
                a1 = plsc.load_gather(tslice(o), [cam])
                a2 = plsc.load_gather(tslice(o + 8), [cam])
                a3 = plsc.load_gather(tslice(o + 16), [cam])
                cx = plsc.load_gather(tslice(o + 24), [cam])
                cy = plsc.load_gather(tslice(o + 32), [cam])
                du = u - cx
                dv = v - cy
                r2 = du * du + dv * dv
                vig = 1.0 + r2 * (a1 + r2 * (a2 + r2 * a3))
                x3.append(xv * vig)
            xr, xg, xb = x3

            def col(k):
                return plsc.load_gather(tslice(COL_OFF + k * NUM_FRAMES),
                                        [frm])
            xr = xr * jnp.exp(col(0))
            xb = xb * jnp.exp(col(1))
            m01, m02, m10, m12, m20, m21 = (col(k) for k in range(2, 8))
            d1 = xg - xr
            d2 = xb - xr
            d3 = xb - xg
            ro = xr + m01 * d1 + m02 * d2
            go = xg - m10 * d1 + m12 * d3
            bo = xb - m20 * d2 - m21 * d3

            outs = []
            for chan, xv in enumerate((ro, go, bo)):
                pp = plsc.load_gather(tslice(DER_OFF + chan * L), [cam])
                qq = plsc.load_gather(tslice(DER_OFF + (3 + chan) * L), [cam])
                cc = plsc.load_gather(tslice(DER_OFF + (6 + chan) * L), [cam])
                dd = plsc.load_gather(tslice(DER_OFF + (9 + chan) * L), [cam])
                xc = jnp.clip(xv, 1e-06, 1.0)
                xc = jnp.exp(cc * _log2(xc))
                outs.append((pp * xc) / (qq * xc + 1.0) + dd)

            plsc.store_scatter(out_v, [jr], outs[0])
            plsc.store_scatter(out_v, [jr1], outs[1])
            plsc.store_scatter(out_v, [jr2], outs[2])

        pltpu.sync_copy(out_v, out_hbm.at[pl.ds(off * 3, CHUNK * 3)])


@jax.jit
def kernel(exposure_params, vignetting_params, color_params, crf_params,
           rgb, pixel_coords, camera_idcs, frame_idcs):
    tab = jnp.concatenate([
        exposure_params,
        jnp.transpose(vignetting_params, (1, 2, 0)).reshape(-1),
        jnp.transpose(color_params, (1, 0)).reshape(-1),
        crf_params.reshape(-1),
    ])
    mesh = plsc.VectorSubcoreMesh(core_axis_name="c", subcore_axis_name="s")
    run = functools.partial(
        pl.kernel,
        mesh=mesh,
        compiler_params=pltpu.CompilerParams(needs_layout_passes=False),
        out_type=jax.ShapeDtypeStruct((3 * N,), jnp.float32),
        scratch_types=[
            pltpu.VMEM((TABV_LEN,), jnp.float32),
            pltpu.VMEM((CHUNK * 3,), jnp.float32),
            pltpu.VMEM((CHUNK * 2,), jnp.float32),
            pltpu.VMEM((CHUNK,), jnp.int32),
            pltpu.VMEM((CHUNK,), jnp.int32),
            pltpu.VMEM((CHUNK * 3,), jnp.float32),
        ],
    )(_kernel_body)
    y = run(tab, rgb.reshape(-1), pixel_coords.reshape(-1),
            camera_idcs, frame_idcs)
    return y.reshape(N, 3)


# speedup vs baseline: 287.0691x; 19.2138x over previous
import functools

import jax
import jax.numpy as jnp
from jax import lax
from jax.experimental import pallas as pl
from jax.experimental.pallas import tpu as pltpu
from jax.experimental.pallas import tpu_sc as plsc

N = 1048576
NUM_FRAMES = 1000
NUM_CAMERAS = 8
FPAD = 1024
INV_W = 1.0 / 1920.0
INV_H = 1.0 / 1080.0
LN2 = 0.6931471805599453

EXPO_OFF = 0
VIG_OFF = FPAD
COL_OFF = VIG_OFF + 15 * NUM_CAMERAS
CRF_OFF = COL_OFF + 8 * FPAD
TAB_LEN = CRF_OFF + NUM_CAMERAS * 12
DER_OFF = TAB_LEN
TABV_LEN = DER_OFF + 12 * 16

NC, NS, L = 2, 16, 16
NW = NC * NS
PPW = N // NW
CHUNK = 8192
NCHUNK = PPW // CHUNK
VPC = CHUNK // 16
UNROLL = 4

_LOG2_C = (
    -7.86056724e-06, 1.44253219e+00, -7.20063736e-01, 4.87638928e-01,
    -3.95413117e-01, 2.48497845e-01,
)
_SQRT2_2_BITS = 0x3F3504F3


def _log2(x):
    bits = plsc.bitcast(x, jnp.int32)
    e = (bits - _SQRT2_2_BITS) >> 23
    m = plsc.bitcast(bits - (e << 23), jnp.float32)
    t = m - 1.0
    p = jnp.full((L,), _LOG2_C[5], jnp.float32)
    for c in _LOG2_C[4::-1]:
        p = p * t + c
    return e.astype(jnp.float32) + p


def _ln(x):
    return _log2(x) * LN2


def _kernel_body(tab_hbm, r_hbm, g_hbm, b_hbm, u_hbm, v_hbm, cam_hbm,
                 frm_hbm, or_hbm, og_hbm, ob_hbm,
                 tab_v, r_v, g_v, b_v, u_v, v_v, cam_v, frm_v,
                 or_v, og_v, ob_v):
    wid = lax.axis_index("s") * NC + lax.axis_index("c")
    base_px = wid * PPW

    lane = lax.broadcasted_iota(jnp.int32, (L,), 0)

    pltpu.sync_copy(tab_hbm, tab_v.at[pl.ds(0, TAB_LEN)])

    def tslice(off):
        return tab_v.at[pl.ds(off, TABV_LEN - off)]

    for i in range(FPAD // L):
        sl = pl.ds(i * L, L)
        ex = tab_v[sl]
        s0 = pl.ds(COL_OFF + i * L, L)
        s1 = pl.ds(COL_OFF + FPAD + i * L, L)
        tab_v[s0] = jnp.exp(tab_v[s0] + ex)
        tab_v[s1] = jnp.exp(tab_v[s1] + ex)
        tab_v[sl] = jnp.exp(ex)

    k4 = lane & 3
    for i in range(6):
        sl = pl.ds(CRF_OFF + i * L, L)
        t = tab_v[sl]
        sp = jnp.maximum(t, 0.0) + _ln(1.0 + jnp.exp(-jnp.abs(t)))
        val = jnp.where(k4 == 2, sp + 0.1, sp + 0.3)
        tab_v[sl] = jnp.where(k4 == 3, t, val)

    cam8 = lane & 7
    for a in range(12):
        kind, chan = a // 3, a % 3
        g_ = (cam8 * 3 + chan) * 4 + CRF_OFF
        if kind == 0:
            val = plsc.load_gather(tab_v, [g_]) * plsc.load_gather(tab_v, [g_ + 1])
        elif kind == 1:
            val = plsc.load_gather(tab_v, [g_]) - 1.0
        elif kind == 2:
            val = plsc.load_gather(tab_v, [g_ + 2]) * LN2
        else:
            val = plsc.load_gather(tab_v, [g_ + 3])
        tab_v[pl.ds(DER_OFF + a * L, L)] = val

    for ch in range(NCHUNK):
        off = base_px + ch * CHUNK
        csl = pl.ds(off, CHUNK)
        pltpu.sync_copy(r_hbm.at[csl], r_v)
        pltpu.sync_copy(g_hbm.at[csl], g_v)
        pltpu.sync_copy(b_hbm.at[csl], b_v)
        pltpu.sync_copy(u_hbm.at[csl], u_v)
        pltpu.sync_copy(v_hbm.at[csl], v_v)
        pltpu.sync_copy(cam_hbm.at[csl], cam_v)
        pltpu.sync_copy(frm_hbm.at[csl], frm_v)

        @plsc.parallel_loop(0, VPC, step=1, unroll=UNROLL)
        def vbody(j):
            vsl = pl.ds(j * L, L)
            cam = cam_v[vsl]
            frm = frm_v[vsl]
            r = r_v[vsl]
            g = g_v[vsl]
            b = b_v[vsl]
            u = u_v[vsl] * INV_W - 0.5
            v = v_v[vsl] * INV_H - 0.5

            wr = plsc.load_gather(tslice(COL_OFF), [frm])
            wg = plsc.load_gather(tab_v, [frm])
            wb_ = plsc.load_gather(tslice(COL_OFF + FPAD), [frm])

            x3 = []
            for chan, (xv, w) in enumerate(((r, wr), (g, wg), (b, wb_))):
                o = VIG_OFF + chan * 5 * 8
                a1 = plsc.load_gather(tslice(o), [cam])
                a2 = plsc.load_gather(tslice(o + 8), [cam])
                a3 = plsc.load_gather(tslice(o + 16), [cam])
                cx = plsc.load_gather(tslice(o + 24), [cam])
                cy = plsc.load_gather(tslice(o + 32), [cam])
                du = u - cx
                dv = v - cy
                r2 = du * du + dv * dv
                vig = 1.0 + r2 * (a1 + r2 * (a2 + r2 * a3))
                x3.append(xv * w * vig)
            xr, xg, xb = x3

            def col(k):
                return plsc.load_gather(tslice(COL_OFF + k * FPAD), [frm])
            m01, m02, m10, m12, m20, m21 = (col(k) for k in range(2, 8))
            d1 = xg - xr
            d2 = xb - xr
            d3 = xb - xg
            ro = xr + m01 * d1 + m02 * d2
            go = xg - m10 * d1 + m12 * d3
            bo = xb - m20 * d2 - m21 * d3

            outs = []
            for chan, xv in enumerate((ro, go, bo)):
                pp = plsc.load_gather(tslice(DER_OFF + chan * L), [cam])
                qq = plsc.load_gather(tslice(DER_OFF + (3 + chan) * L), [cam])
                cc = plsc.load_gather(tslice(DER_OFF + (6 + chan) * L), [cam])
                dd = plsc.load_gather(tslice(DER_OFF + (9 + chan) * L), [cam])
                xc = jnp.clip(xv, 1e-06, 1.0)
                xc = jnp.exp(cc * _log2(xc))
                outs.append((pp * xc) / (qq * xc + 1.0) + dd)

            or_v[vsl] = outs[0]
            og_v[vsl] = outs[1]
            ob_v[vsl] = outs[2]

        pltpu.sync_copy(or_v, or_hbm.at[csl])
        pltpu.sync_copy(og_v, og_hbm.at[csl])
        pltpu.sync_copy(ob_v, ob_hbm.at[csl])


@jax.jit
def kernel(exposure_params, vignetting_params, color_params, crf_params,
           rgb, pixel_coords, camera_idcs, frame_idcs):
    pad = FPAD - NUM_FRAMES
    tab = jnp.concatenate([
        jnp.pad(exposure_params, (0, pad)),
        jnp.transpose(vignetting_params, (1, 2, 0)).reshape(-1),
        jnp.pad(jnp.transpose(color_params, (1, 0)), ((0, 0), (0, pad))
                ).reshape(-1),
        crf_params.reshape(-1),
    ])
    mesh = plsc.VectorSubcoreMesh(core_axis_name="c", subcore_axis_name="s")
    f32 = jnp.float32
    run = functools.partial(
        pl.kernel,
        mesh=mesh,
        compiler_params=pltpu.CompilerParams(needs_layout_passes=False),
        out_type=(
            jax.ShapeDtypeStruct((N,), f32),
            jax.ShapeDtypeStruct((N,), f32),
            jax.ShapeDtypeStruct((N,), f32),
        ),
        scratch_types=[
            pltpu.VMEM((TABV_LEN,), f32),
            pltpu.VMEM((CHUNK,), f32),
            pltpu.VMEM((CHUNK,), f32),
            pltpu.VMEM((CHUNK,), f32),
            pltpu.VMEM((CHUNK,), f32),
            pltpu.VMEM((CHUNK,), f32),
            pltpu.VMEM((CHUNK,), jnp.int32),
            pltpu.VMEM((CHUNK,), jnp.int32),
            pltpu.VMEM((CHUNK,), f32),
            pltpu.VMEM((CHUNK,), f32),
            pltpu.VMEM((CHUNK,), f32),
        ],
    )(_kernel_body)
    yr, yg, yb = run(tab, rgb[:, 0], rgb[:, 1], rgb[:, 2],
                     pixel_coords[:, 0], pixel_coords[:, 1],
                     camera_idcs, frame_idcs)
    return jnp.stack([yr, yg, yb], axis=-1)


# speedup vs baseline: 329.9343x; 1.1493x over previous
import functools

import jax
import jax.numpy as jnp
from jax import lax
from jax.experimental import pallas as pl
from jax.experimental.pallas import tpu as pltpu
from jax.experimental.pallas import tpu_sc as plsc

N = 1048576
NUM_FRAMES = 1000
NUM_CAMERAS = 8
FPAD = 1024
INV_W = 1.0 / 1920.0
INV_H = 1.0 / 1080.0
LN2 = 0.6931471805599453

EXPO_OFF = 0
VIG_OFF = FPAD
COL_OFF = VIG_OFF + 15 * NUM_CAMERAS
CRF_OFF = COL_OFF + 8 * FPAD
TAB_LEN = CRF_OFF + NUM_CAMERAS * 12
DER_OFF = TAB_LEN
TABV_LEN = DER_OFF + 12 * 16

NC, NS, L = 2, 16, 16
NW = NC * NS
PPW = N // NW
CHUNK = 4096
NCHUNK = PPW // CHUNK
VPC = CHUNK // 16
UNROLL = 4

_LOG2_C = (
    -7.86056724e-06, 1.44253219e+00, -7.20063736e-01, 4.87638928e-01,
    -3.95413117e-01, 2.48497845e-01,
)
_SQRT2_2_BITS = 0x3F3504F3


def _log2(x):
    bits = plsc.bitcast(x, jnp.int32)
    e = (bits - _SQRT2_2_BITS) >> 23
    m = plsc.bitcast(bits - (e << 23), jnp.float32)
    t = m - 1.0
    p = jnp.full((L,), _LOG2_C[5], jnp.float32)
    for c in _LOG2_C[4::-1]:
        p = p * t + c
    return e.astype(jnp.float32) + p


def _ln(x):
    return _log2(x) * LN2


def _kernel_body(tab_hbm, r_hbm, g_hbm, b_hbm, u_hbm, v_hbm, cf_hbm,
                 or_hbm, og_hbm, ob_hbm,
                 tab_v, r_v, g_v, b_v, u_v, v_v, cf_v,
                 or_v, og_v, ob_v, in_sems, out_sems):
    wid = lax.axis_index("s") * NC + lax.axis_index("c")
    base_px = wid * PPW

    lane = lax.broadcasted_iota(jnp.int32, (L,), 0)

    pltpu.sync_copy(tab_hbm, tab_v.at[pl.ds(0, TAB_LEN)])

    def tslice(off):
        return tab_v.at[pl.ds(off, TABV_LEN - off)]

    for i in range(FPAD // L):
        sl = pl.ds(i * L, L)
        ex = tab_v[sl]
        s0 = pl.ds(COL_OFF + i * L, L)
        s1 = pl.ds(COL_OFF + FPAD + i * L, L)
        tab_v[s0] = jnp.exp(tab_v[s0] + ex)
        tab_v[s1] = jnp.exp(tab_v[s1] + ex)
        tab_v[sl] = jnp.exp(ex)

    k4 = lane & 3
    for i in range(6):
        sl = pl.ds(CRF_OFF + i * L, L)
        t = tab_v[sl]
        sp = jnp.maximum(t, 0.0) + _ln(1.0 + jnp.exp(-jnp.abs(t)))
        val = jnp.where(k4 == 2, sp + 0.1, sp + 0.3)
        tab_v[sl] = jnp.where(k4 == 3, t, val)

    cam8 = lane & 7
    for a in range(12):
        kind, chan = a // 3, a % 3
        g_ = (cam8 * 3 + chan) * 4 + CRF_OFF
        if kind == 0:
            val = plsc.load_gather(tab_v, [g_]) * plsc.load_gather(tab_v, [g_ + 1])
        elif kind == 1:
            val = plsc.load_gather(tab_v, [g_]) - 1.0
        elif kind == 2:
            val = plsc.load_gather(tab_v, [g_ + 2]) * LN2
        else:
            val = plsc.load_gather(tab_v, [g_ + 3])
        tab_v[pl.ds(DER_OFF + a * L, L)] = val

    def start_in(ch, s):
        csl = pl.ds(base_px + ch * CHUNK, CHUNK)
        sem = in_sems.at[s]
        bsl = pl.ds(s * CHUNK, CHUNK)
        return [
            pltpu.async_copy(r_hbm.at[csl], r_v.at[bsl], sem),
            pltpu.async_copy(g_hbm.at[csl], g_v.at[bsl], sem),
            pltpu.async_copy(b_hbm.at[csl], b_v.at[bsl], sem),
            pltpu.async_copy(u_hbm.at[csl], u_v.at[bsl], sem),
            pltpu.async_copy(v_hbm.at[csl], v_v.at[bsl], sem),
            pltpu.async_copy(cf_hbm.at[csl], cf_v.at[bsl], sem),
        ]

    in_handles = {0: start_in(0, 0)}
    out_handles = {}

    for ch in range(NCHUNK):
        s = ch & 1
        if ch + 1 < NCHUNK:
            in_handles[ch + 1] = start_in(ch + 1, 1 - s)
        for h in in_handles.pop(ch):
            h.wait()
        if ch - 2 in out_handles:
            for h in out_handles.pop(ch - 2):
                h.wait()

        bsl = pl.ds(s * CHUNK, CHUNK)
        rb, gb, bb_, ub, vb, cfb = (x.at[bsl] for x in
                                    (r_v, g_v, b_v, u_v, v_v, cf_v))
        orb, ogb, obb = (x.at[bsl] for x in (or_v, og_v, ob_v))

        @plsc.parallel_loop(0, VPC, step=1, unroll=UNROLL)
        def vbody(j):
            vsl = pl.ds(j * L, L)
            cf = cfb[vsl]
            cam = cf & 7
            frm = cf >> 3
            r = rb[vsl]
            g = gb[vsl]
            b = bb_[vsl]
            u = ub[vsl] * INV_W - 0.5
            v = vb[vsl] * INV_H - 0.5

            wr = plsc.load_gather(tslice(COL_OFF), [frm])
            wg = plsc.load_gather(tab_v, [frm])
            wb_ = plsc.load_gather(tslice(COL_OFF + FPAD), [frm])

            x3 = []
            for chan, (xv, w) in enumerate(((r, wr), (g, wg), (b, wb_))):
                o = VIG_OFF + chan * 5 * 8
                a1 = plsc.load_gather(tslice(o), [cam])
                a2 = plsc.load_gather(tslice(o + 8), [cam])
                a3 = plsc.load_gather(tslice(o + 16), [cam])
                cx = plsc.load_gather(tslice(o + 24), [cam])
                cy = plsc.load_gather(tslice(o + 32), [cam])
                du = u - cx
                dv = v - cy
                r2 = du * du + dv * dv
                vig = 1.0 + r2 * (a1 + r2 * (a2 + r2 * a3))
                x3.append(xv * w * vig)
            xr, xg, xb = x3

            def col(k):
                return plsc.load_gather(tslice(COL_OFF + k * FPAD), [frm])
            m01, m02, m10, m12, m20, m21 = (col(k) for k in range(2, 8))
            d1 = xg - xr
            d2 = xb - xr
            d3 = xb - xg
            ro = xr + m01 * d1 + m02 * d2
            go = xg - m10 * d1 + m12 * d3
            bo = xb - m20 * d2 - m21 * d3

            outs = []
            for chan, xv in enumerate((ro, go, bo)):
                pp = plsc.load_gather(tslice(DER_OFF + chan * L), [cam])
                qq = plsc.load_gather(tslice(DER_OFF + (3 + chan) * L), [cam])
                cc = plsc.load_gather(tslice(DER_OFF + (6 + chan) * L), [cam])
                dd = plsc.load_gather(tslice(DER_OFF + (9 + chan) * L), [cam])
                xc = jnp.clip(xv, 1e-06, 1.0)
                xc = jnp.exp(cc * _log2(xc))
                outs.append((pp * xc) / (qq * xc + 1.0) + dd)

            orb[vsl] = outs[0]
            ogb[vsl] = outs[1]
            obb[vsl] = outs[2]

        csl = pl.ds(base_px + ch * CHUNK, CHUNK)
        sem = out_sems.at[s]
        out_handles[ch] = [
            pltpu.async_copy(orb, or_hbm.at[csl], sem),
            pltpu.async_copy(ogb, og_hbm.at[csl], sem),
            pltpu.async_copy(obb, ob_hbm.at[csl], sem),
        ]

    for ch in sorted(out_handles):
        for h in out_handles.pop(ch):
            h.wait()


@jax.jit
def kernel(exposure_params, vignetting_params, color_params, crf_params,
           rgb, pixel_coords, camera_idcs, frame_idcs):
    pad = FPAD - NUM_FRAMES
    tab = jnp.concatenate([
        jnp.pad(exposure_params, (0, pad)),
        jnp.transpose(vignetting_params, (1, 2, 0)).reshape(-1),
        jnp.pad(jnp.transpose(color_params, (1, 0)), ((0, 0), (0, pad))
                ).reshape(-1),
        crf_params.reshape(-1),
    ])
    cf = (frame_idcs << 3) | camera_idcs
    mesh = plsc.VectorSubcoreMesh(core_axis_name="c", subcore_axis_name="s")
    f32 = jnp.float32
    run = functools.partial(
        pl.kernel,
        mesh=mesh,
        compiler_params=pltpu.CompilerParams(needs_layout_passes=False),
        out_type=(
            jax.ShapeDtypeStruct((N,), f32),
            jax.ShapeDtypeStruct((N,), f32),
            jax.ShapeDtypeStruct((N,), f32),
        ),
        scratch_types=[
            pltpu.VMEM((TABV_LEN,), f32),
            pltpu.VMEM((2 * CHUNK,), f32),
            pltpu.VMEM((2 * CHUNK,), f32),
            pltpu.VMEM((2 * CHUNK,), f32),
            pltpu.VMEM((2 * CHUNK,), f32),
            pltpu.VMEM((2 * CHUNK,), f32),
            pltpu.VMEM((2 * CHUNK,), jnp.int32),
            pltpu.VMEM((2 * CHUNK,), f32),
            pltpu.VMEM((2 * CHUNK,), f32),
            pltpu.VMEM((2 * CHUNK,), f32),
            pltpu.SemaphoreType.DMA((2,)),
            pltpu.SemaphoreType.DMA((2,)),
        ],
    )(_kernel_body)
    yr, yg, yb = run(tab, rgb[:, 0], rgb[:, 1], rgb[:, 2],
                     pixel_coords[:, 0], pixel_coords[:, 1], cf)
    return jnp.stack([yr, yg, yb], axis=-1)


# speedup vs baseline: 340.3867x; 1.0317x over previous
import functools

import jax
import jax.numpy as jnp
from jax import lax
from jax.experimental import pallas as pl
from jax.experimental.pallas import tpu as pltpu
from jax.experimental.pallas import tpu_sc as plsc

N = 1048576
NUM_FRAMES = 1000
NUM_CAMERAS = 8
FPAD = 1024
INV_W = 1.0 / 1920.0
INV_H = 1.0 / 1080.0
LN2 = 0.6931471805599453

EXPO_OFF = 0
VIG_OFF = FPAD
COL_OFF = VIG_OFF + 128
CRF_OFF = COL_OFF + 5 * FPAD
TAB_LEN = CRF_OFF + NUM_CAMERAS * 12
DER_OFF = TAB_LEN
TABV_LEN = DER_OFF + 12 * 16

NC, NS, L = 2, 16, 16
NW = NC * NS
PPW = N // NW
CHUNK = 4096
NCHUNK = PPW // CHUNK
VPC = CHUNK // 16
UNROLL = 4

_LOG2_C = (
    5.72799704e-05, 1.44173062e+00, -7.26574968e-01, 5.17322850e-01,
    -3.20043508e-01,
)
_SQRT2_2_BITS = 0x3F3504F3


def _log2(x):
    bits = plsc.bitcast(x, jnp.int32)
    e = (bits - _SQRT2_2_BITS) >> 23
    m = plsc.bitcast(bits - (e << 23), jnp.float32)
    t = m - 1.0
    p = jnp.full((L,), _LOG2_C[4], jnp.float32)
    for c in _LOG2_C[3::-1]:
        p = p * t + c
    return e.astype(jnp.float32) + p


def _ln(x):
    return _log2(x) * LN2


def _kernel_body(tab_hbm, r_hbm, g_hbm, b_hbm, u_hbm, v_hbm, cf_hbm,
                 or_hbm, og_hbm, ob_hbm,
                 tab_v, r_v, g_v, b_v, u_v, v_v, cf_v,
                 or_v, og_v, ob_v, in_sems, out_sems):
    wid = lax.axis_index("s") * NC + lax.axis_index("c")
    base_px = wid * PPW

    lane = lax.broadcasted_iota(jnp.int32, (L,), 0)

    pltpu.sync_copy(tab_hbm, tab_v.at[pl.ds(0, TAB_LEN)])

    def tslice(off):
        return tab_v.at[pl.ds(off, TABV_LEN - off)]

    for i in range(FPAD // L):
        sl = pl.ds(i * L, L)
        ex = tab_v[sl]
        s0 = pl.ds(COL_OFF + i * L, L)
        s1 = pl.ds(COL_OFF + FPAD + i * L, L)
        tab_v[s0] = jnp.exp(tab_v[s0] + ex)
        tab_v[s1] = jnp.exp(tab_v[s1] + ex)
        tab_v[sl] = jnp.exp(ex)

    k4 = lane & 3
    for i in range(6):
        sl = pl.ds(CRF_OFF + i * L, L)
        t = tab_v[sl]
        sp = jnp.maximum(t, 0.0) + _ln(1.0 + jnp.exp(-jnp.abs(t)))
        val = jnp.where(k4 == 2, sp + 0.1, sp + 0.3)
        tab_v[sl] = jnp.where(k4 == 3, t, val)

    cam8 = lane & 7
    for a in range(12):
        kind, chan = a // 3, a % 3
        g_ = (cam8 * 3 + chan) * 4 + CRF_OFF
        if kind == 0:
            val = plsc.load_gather(tab_v, [g_]) * plsc.load_gather(tab_v, [g_ + 1])
        elif kind == 1:
            val = plsc.load_gather(tab_v, [g_]) - 1.0
        elif kind == 2:
            val = plsc.load_gather(tab_v, [g_ + 2]) * LN2
        else:
            val = plsc.load_gather(tab_v, [g_ + 3])
        tab_v[pl.ds(DER_OFF + a * L, L)] = val

    def start_in(ch, s):
        csl = pl.ds(base_px + ch * CHUNK, CHUNK)
        sem = in_sems.at[s]
        bsl = pl.ds(s * CHUNK, CHUNK)
        return [
            pltpu.async_copy(r_hbm.at[csl], r_v.at[bsl], sem),
            pltpu.async_copy(g_hbm.at[csl], g_v.at[bsl], sem),
            pltpu.async_copy(b_hbm.at[csl], b_v.at[bsl], sem),
            pltpu.async_copy(u_hbm.at[csl], u_v.at[bsl], sem),
            pltpu.async_copy(v_hbm.at[csl], v_v.at[bsl], sem),
            pltpu.async_copy(cf_hbm.at[csl], cf_v.at[bsl], sem),
        ]

    in_handles = {0: start_in(0, 0)}
    out_handles = {}

    for ch in range(NCHUNK):
        s = ch & 1
        if ch + 1 < NCHUNK:
            in_handles[ch + 1] = start_in(ch + 1, 1 - s)
        for h in in_handles.pop(ch):
            h.wait()
        if ch - 2 in out_handles:
            for h in out_handles.pop(ch - 2):
                h.wait()

        bsl = pl.ds(s * CHUNK, CHUNK)
        rb, gb, bb_, ub, vb, cfb = (x.at[bsl] for x in
                                    (r_v, g_v, b_v, u_v, v_v, cf_v))
        orb, ogb, obb = (x.at[bsl] for x in (or_v, og_v, ob_v))

        @plsc.parallel_loop(0, VPC, step=1, unroll=UNROLL)
        def vbody(j):
            vsl = pl.ds(j * L, L)
            cf = cfb[vsl]
            cam = cf & 7
            frm = cf >> 3
            r = rb[vsl]
            g = gb[vsl]
            b = bb_[vsl]
            u = ub[vsl] * INV_W - 0.5
            v = vb[vsl] * INV_H - 0.5

            wr = plsc.load_gather(tslice(COL_OFF), [frm])
            wg = plsc.load_gather(tab_v, [frm])
            wb_ = plsc.load_gather(tslice(COL_OFF + FPAD), [frm])

            x3 = []
            for chan, (xv, w) in enumerate(((r, wr), (g, wg), (b, wb_))):
                o = VIG_OFF + chan * 5 * 8
                a1 = plsc.load_gather(tslice(o), [cam])
                a2 = plsc.load_gather(tslice(o + 8), [cam])
                a3 = plsc.load_gather(tslice(o + 16), [cam])
                cx = plsc.load_gather(tslice(o + 24), [cam])
                cy = plsc.load_gather(tslice(o + 32), [cam])
                du = u - cx
                dv = v - cy
                r2 = du * du + dv * dv
                vig = 1.0 + r2 * (a1 + r2 * (a2 + r2 * a3))
                x3.append(xv * w * vig)
            xr, xg, xb = x3

            def colpair(k):
                w = plsc.bitcast(
                    plsc.load_gather(tslice(COL_OFF + k * FPAD), [frm]),
                    jnp.int32)
                hi = plsc.bitcast(w & jnp.int32(-65536), jnp.float32)
                lo = plsc.bitcast(w << 16, jnp.float32)
                return hi, lo
            m01, m02 = colpair(2)
            m10, m12 = colpair(3)
            m20, m21 = colpair(4)
            d1 = xg - xr
            d2 = xb - xr
            d3 = xb - xg
            ro = xr + m01 * d1 + m02 * d2
            go = xg - m10 * d1 + m12 * d3
            bo = xb - m20 * d2 - m21 * d3

            outs = []
            for chan, xv in enumerate((ro, go, bo)):
                pp = plsc.load_gather(tslice(DER_OFF + chan * L), [cam])
                qq = plsc.load_gather(tslice(DER_OFF + (3 + chan) * L), [cam])
                cc = plsc.load_gather(tslice(DER_OFF + (6 + chan) * L), [cam])
                dd = plsc.load_gather(tslice(DER_OFF + (9 + chan) * L), [cam])
                xc = jnp.clip(xv, 1e-06, 1.0)
                xc = jnp.exp(cc * _log2(xc))
                outs.append((pp * xc) / (qq * xc + 1.0) + dd)

            orb[vsl] = outs[0]
            ogb[vsl] = outs[1]
            obb[vsl] = outs[2]

        csl = pl.ds(base_px + ch * CHUNK, CHUNK)
        sem = out_sems.at[s]
        out_handles[ch] = [
            pltpu.async_copy(orb, or_hbm.at[csl], sem),
            pltpu.async_copy(ogb, og_hbm.at[csl], sem),
            pltpu.async_copy(obb, ob_hbm.at[csl], sem),
        ]

    for ch in sorted(out_handles):
        for h in out_handles.pop(ch):
            h.wait()


@jax.jit
def kernel(exposure_params, vignetting_params, color_params, crf_params,
           rgb, pixel_coords, camera_idcs, frame_idcs):
    pad = FPAD - NUM_FRAMES
    col_t = jnp.pad(jnp.transpose(color_params, (1, 0)), ((0, 0), (0, pad)))

    def bf16pack(a, b):
        ai = jax.lax.bitcast_convert_type(a, jnp.int32) + 0x8000
        bi = jax.lax.bitcast_convert_type(b, jnp.int32) + 0x8000
        w = (ai & jnp.int32(-65536)) | ((bi >> 16) & 0xFFFF)
        return jax.lax.bitcast_convert_type(w, jnp.float32)

    tab = jnp.concatenate([
        jnp.pad(exposure_params, (0, pad)),
        jnp.pad(jnp.transpose(vignetting_params, (1, 2, 0)).reshape(-1),
                (0, 8)),
        col_t[0], col_t[1],
        bf16pack(col_t[2], col_t[3]),
        bf16pack(col_t[4], col_t[5]),
        bf16pack(col_t[6], col_t[7]),
        crf_params.reshape(-1),
    ])
    cf = (frame_idcs << 3) | camera_idcs
    mesh = plsc.VectorSubcoreMesh(core_axis_name="c", subcore_axis_name="s")
    f32 = jnp.float32
    run = functools.partial(
        pl.kernel,
        mesh=mesh,
        compiler_params=pltpu.CompilerParams(needs_layout_passes=False),
        out_type=(
            jax.ShapeDtypeStruct((N,), f32),
            jax.ShapeDtypeStruct((N,), f32),
            jax.ShapeDtypeStruct((N,), f32),
        ),
        scratch_types=[
            pltpu.VMEM((TABV_LEN,), f32),
            pltpu.VMEM((2 * CHUNK,), f32),
            pltpu.VMEM((2 * CHUNK,), f32),
            pltpu.VMEM((2 * CHUNK,), f32),
            pltpu.VMEM((2 * CHUNK,), f32),
            pltpu.VMEM((2 * CHUNK,), f32),
            pltpu.VMEM((2 * CHUNK,), jnp.int32),
            pltpu.VMEM((2 * CHUNK,), f32),
            pltpu.VMEM((2 * CHUNK,), f32),
            pltpu.VMEM((2 * CHUNK,), f32),
            pltpu.SemaphoreType.DMA((2,)),
            pltpu.SemaphoreType.DMA((2,)),
        ],
    )(_kernel_body)
    yr, yg, yb = run(tab, rgb[:, 0], rgb[:, 1], rgb[:, 2],
                     pixel_coords[:, 0], pixel_coords[:, 1], cf)
    return jnp.stack([yr, yg, yb], axis=-1)
